# Initial kernel scaffold; baseline (speedup 1.0000x reference)
import jax
import jax.numpy as jnp
from jax.experimental import pallas as pl


def kernel(node_ids, node_timestamp, edge_index, edge_strength, cu_seqlens, emb_table, deg_in_table, deg_out_table, graph_token, Wq, Wk, Wv, Wo, W1, b1, W2, b2, Wf, bf, Wp, bp):
    raise NotImplementedError("write your pallas kernel here")


# speedup vs baseline: 1.7864x; 1.7864x over previous
import functools

import jax
import jax.numpy as jnp
import numpy as np
from jax import lax
from jax.experimental import pallas as pl
from jax.experimental.pallas import tpu as pltpu
from jax.experimental.pallas import tpu_sc as plsc

B = 4
NPAD = 1024
T = NPAD + 1
D = 256
H = 8
DH = D // H
DEG = 64
EPG = 8192
FF = 1024
TPAD = 1152
NEG = -1e30

NC = 2
NS = 16
NW = NC * NS
RPW = (B * NPAD) // NW
EPW = (B * EPG) // NW
ES_HALF = 2 * NPAD * NPAD
ZCH = 16384


def _sc_sparse(node_ids, edge_index, edge_strength, emb_table):
    mesh = plsc.VectorSubcoreMesh(core_axis_name="c", subcore_axis_name="s")
    out_type = (
        jax.ShapeDtypeStruct((B * NPAD, D), jnp.float32),
        jax.ShapeDtypeStruct((B * NPAD * NPAD,), jnp.float32),
    )

    @functools.partial(
        pl.kernel,
        out_type=out_type,
        mesh=mesh,
        scratch_types=[
            pltpu.VMEM((RPW,), jnp.int32),
            pltpu.VMEM((RPW, D), jnp.float32),
            pltpu.VMEM((EPW,), jnp.int32),
            pltpu.VMEM((EPW,), jnp.int32),
            pltpu.VMEM((EPW,), jnp.float32),
            pltpu.VMEM((8, 128), jnp.int32),
            pltpu.VMEM((8, 128), jnp.float32),
            pltpu.VMEM((ZCH,), jnp.float32),
        ],
    )
    def k(ids_hbm, ei_hbm, estr_hbm, emb_hbm,
          nf_hbm, es_hbm,
          gidx_v, grow_v, src_v, dst_v, str_v, ei2_v, ev2_v, zero_v):
        c = lax.axis_index("c")
        s = lax.axis_index("s")
        wid = c * NS + s
        g_local = s // 8
        gid = 2 * c + g_local

        gbase = wid * RPW
        pltpu.sync_copy(ids_hbm.at[pl.ds(gbase, RPW)], gidx_v)
        pltpu.sync_copy(emb_hbm.at[gidx_v], grow_v)
        pltpu.sync_copy(grow_v, nf_hbm.at[pl.ds(gbase, RPW)])

        @pl.loop(0, ZCH, step=16)
        def _(i):
            zero_v[pl.ds(i, 16)] = jnp.zeros((16,), jnp.float32)

        es_tile_base = c * ES_HALF + s * (ES_HALF // NS)

        @pl.loop(0, ES_HALF // NS, step=ZCH)
        def _(off):
            pltpu.sync_copy(zero_v, es_hbm.at[pl.ds(es_tile_base + off, ZCH)])

        ebase = c * (2 * EPG) + s * EPW
        pltpu.sync_copy(ei_hbm.at[0, pl.ds(ebase, EPW)], src_v)
        pltpu.sync_copy(ei_hbm.at[1, pl.ds(ebase, EPW)], dst_v)
        pltpu.sync_copy(estr_hbm.at[pl.ds(ebase, EPW)], str_v)

        es_off = gid * (NPAD * NPAD)

        @pl.loop(0, 8)
        def _(j):
            @pl.loop(0, 8)
            def _(kk):
                col = kk * 16
                sl = pl.ds(j * 128 + col, 16)
                ei2_v[j, pl.ds(col, 16)] = (src_v[sl] * NPAD + dst_v[sl]
                                            + es_off)
                ev2_v[j, pl.ds(col, 16)] = str_v[sl]

        plsc.subcore_barrier()

        @pl.loop(0, 8)
        def _(j):
            pltpu.sync_copy(ev2_v.at[j], es_hbm.at[ei2_v.at[j]])

    return k(node_ids, edge_index, edge_strength, emb_table)


def _tc_body(nf_ref, srcb_ref, dstb_ref, ts_ref, es_ref, din_ref, dout_ref,
             gt_ref, wq_ref, wk_ref, wv_ref, wo_ref, w1_ref, b1_ref, w2_ref,
             b2_ref, wf_ref, bfv_ref, wp_ref, bp_ref, out_ref):
    f32 = jnp.float32
    dot = functools.partial(jnp.dot, preferred_element_type=f32)

    ones8 = jnp.ones((8, EPG), f32)
    lane = lax.broadcasted_iota(jnp.int32, (1, 128), 1)

    def counts_col(xb):
        chunks = []
        for cb in range(NPAD // 128):
            oh = (xb == lane + cb * 128).astype(f32)
            chunks.append(dot(ones8, oh))
        c8 = jnp.concatenate(chunks, axis=1)
        return jnp.transpose(c8)[:, 0:1]

    def bucket_onehot(ccol):
        cf = jnp.maximum(ccol, 1.0)
        lg = (lax.bitcast_convert_type(cf, jnp.int32) >> 23) - 127
        bk = jnp.clip(lg, 0, DEG - 1)
        return (bk == lax.broadcasted_iota(jnp.int32, (1, DEG), 1)
                ).astype(f32)

    deg_in = dot(bucket_onehot(counts_col(dstb_ref[0])), din_ref[...])
    deg_out = dot(bucket_onehot(counts_col(srcb_ref[0])), dout_ref[...])

    h_nodes = nf_ref[0] + deg_in + deg_out
    h = jnp.concatenate(
        [h_nodes, gt_ref[...], jnp.zeros((TPAD - T, D), f32)], axis=0)

    q = dot(h, wq_ref[...])
    km = dot(h, wk_ref[...])
    v = dot(h, wv_ref[...])

    lane = lax.broadcasted_iota(jnp.int32, (1, 128), 1)
    right = jnp.where(lane == 0, 0.0, NEG).astype(f32)
    top = jnp.concatenate(
        [es_ref[0], jnp.broadcast_to(right, (NPAD, 128))], axis=1)
    lane_full = lax.broadcasted_iota(jnp.int32, (1, TPAD), 1)
    bot_row = jnp.where(lane_full <= NPAD, 0.0, NEG).astype(f32)
    bias = jnp.concatenate(
        [top, jnp.broadcast_to(bot_row, (TPAD - NPAD, TPAD))], axis=0)

    scale = f32(1.0 / np.sqrt(DH))
    heads = []
    for i in range(H):
        qh = q[:, i * DH:(i + 1) * DH] * scale
        kh = km[:, i * DH:(i + 1) * DH]
        logits = lax.dot_general(qh, kh, (((1,), (1,)), ((), ())),
                                 preferred_element_type=f32) + bias
        m = jnp.max(logits, axis=1, keepdims=True)
        p = jnp.exp(logits - m)
        attn = p / jnp.sum(p, axis=1, keepdims=True)
        heads.append(dot(attn, v[:, i * DH:(i + 1) * DH]))
    o = jnp.concatenate(heads, axis=1)

    o2 = dot(o, wo_ref[...]) + h
    ffh = jnp.maximum(dot(o2, w1_ref[...]) + b1_ref[...], 0.0)
    ffo = dot(ffh, w2_ref[...]) + b2_ref[...] + o2

    nz = jnp.sum((ts_ref[0, 0, :] != 0).astype(jnp.int32))
    ridx = lax.broadcasted_iota(jnp.int32, (TPAD, 1), 0)
    sel = jnp.where(ridx + 1 < nz, ffo, 0.0)
    h0 = (jnp.sum(sel, axis=0, keepdims=True)
          / jnp.maximum(nz, 1).astype(f32))
    hs = h0 + jnp.tanh(dot(h0, wf_ref[...]) + bfv_ref[...])

    graph_rep = ffo[NPAD:NPAD + 1, :]
    s0 = jnp.sum(graph_rep * wp_ref[0:1, :])
    s1 = jnp.sum(hs * wp_ref[1:2, :])
    pred = jnp.maximum(s0 + s1 + bp_ref[0, 0], 0.0)
    out_ref[0, 0, :] = jnp.full((128,), pred, f32)


def _tc_forward(nf, srcb, dstb, ts3, es, deg_in_table, deg_out_table, gt2,
                Wq, Wk, Wv, Wo, W1, b1_2, W2, b2_2, Wf, bf_2, Wp2, bp2):
    full = lambda shape: pl.BlockSpec(shape, lambda b: (0,) * len(shape))
    perb3 = lambda s1_, s2_: pl.BlockSpec((1, s1_, s2_), lambda b: (b, 0, 0))
    out = pl.pallas_call(
        _tc_body,
        grid=(B,),
        in_specs=[
            perb3(NPAD, D),
            perb3(EPG, 128),
            perb3(EPG, 128),
            perb3(1, NPAD),
            perb3(NPAD, NPAD),
            full((DEG, D)),
            full((DEG, D)),
            full((1, D)),
            full((D, D)), full((D, D)), full((D, D)), full((D, D)),
            full((D, FF)), full((1, FF)), full((FF, D)), full((1, D)),
            full((D, D)), full((1, D)),
            full((2, D)),
            full((1, 128)),
        ],
        out_specs=pl.BlockSpec((1, 1, 128), lambda b: (b, 0, 0)),
        out_shape=jax.ShapeDtypeStruct((B, 1, 128), jnp.float32),
        compiler_params=pltpu.CompilerParams(
            dimension_semantics=("arbitrary",)),
    )(nf, srcb, dstb, ts3, es, deg_in_table, deg_out_table, gt2,
      Wq, Wk, Wv, Wo, W1, b1_2, W2, b2_2, Wf, bf_2, Wp2, bp2)
    return out[:, 0, :1]


def kernel(node_ids, node_timestamp, edge_index, edge_strength, cu_seqlens,
           emb_table, deg_in_table, deg_out_table, graph_token,
           Wq, Wk, Wv, Wo, W1, b1, W2, b2, Wf, bf, Wp, bp):
    node_ids = node_ids.astype(jnp.int32)
    edge_index = edge_index.astype(jnp.int32)

    nf, es = _sc_sparse(node_ids, edge_index, edge_strength, emb_table)

    srcb = jnp.broadcast_to(edge_index[0].reshape(B, EPG, 1), (B, EPG, 128))
    dstb = jnp.broadcast_to(edge_index[1].reshape(B, EPG, 1), (B, EPG, 128))

    return _tc_forward(
        nf.reshape(B, NPAD, D),
        srcb, dstb,
        node_timestamp.reshape(B, 1, NPAD),
        es.reshape(B, NPAD, NPAD),
        deg_in_table, deg_out_table,
        graph_token.reshape(1, D),
        Wq, Wk, Wv, Wo,
        W1, b1.reshape(1, FF), W2, b2.reshape(1, D),
        Wf, bf.reshape(1, D),
        Wp.reshape(2, D),
        jnp.broadcast_to(bp.reshape(1, 1), (1, 128)),
    )


# speedup vs baseline: 2.0432x; 1.1438x over previous
import functools

import jax
import jax.numpy as jnp
import numpy as np
from jax import lax
from jax.experimental import pallas as pl
from jax.experimental.pallas import tpu as pltpu
from jax.experimental.pallas import tpu_sc as plsc

B = 4
NPAD = 1024
T = NPAD + 1
D = 256
H = 8
DH = D // H
DEG = 64
EPG = 8192
FF = 1024
TPAD = 1152
NEG = -1e30

NC = 2
NS = 16
NW = NC * NS
RPW = (B * NPAD) // NW
EPW = (B * EPG) // NW
ES_HALF = 2 * NPAD * NPAD
ZCH = 16384


def _sc_sparse(node_ids, edge_index, edge_strength, emb_table):
    mesh = plsc.VectorSubcoreMesh(core_axis_name="c", subcore_axis_name="s")
    out_type = (
        jax.ShapeDtypeStruct((B * NPAD, D), jnp.float32),
        jax.ShapeDtypeStruct((B * NPAD * NPAD,), jnp.float32),
    )

    @functools.partial(
        pl.kernel,
        out_type=out_type,
        mesh=mesh,
        scratch_types=[
            pltpu.VMEM((RPW,), jnp.int32),
            pltpu.VMEM((RPW, D), jnp.float32),
            pltpu.VMEM((EPW,), jnp.int32),
            pltpu.VMEM((EPW,), jnp.int32),
            pltpu.VMEM((EPW,), jnp.float32),
            pltpu.VMEM((8, 128), jnp.int32),
            pltpu.VMEM((8, 128), jnp.float32),
            pltpu.VMEM((ZCH,), jnp.float32),
            pltpu.SemaphoreType.DMA,
        ],
    )
    def k(ids_hbm, ei_hbm, estr_hbm, emb_hbm,
          nf_hbm, es_hbm,
          gidx_v, grow_v, src_v, dst_v, str_v, ei2_v, ev2_v, zero_v, sem):
        c = lax.axis_index("c")
        s = lax.axis_index("s")
        wid = c * NS + s
        g_local = s // 8
        gid = 2 * c + g_local

        @pl.loop(0, ZCH, step=16)
        def _(i):
            zero_v[pl.ds(i, 16)] = jnp.zeros((16,), jnp.float32)

        es_tile_base = c * ES_HALF + s * (ES_HALF // NS)
        zcps = [
            pltpu.async_copy(
                zero_v, es_hbm.at[pl.ds(es_tile_base + off, ZCH)], sem)
            for off in range(0, ES_HALF // NS, ZCH)
        ]

        gbase = wid * RPW
        pltpu.sync_copy(ids_hbm.at[pl.ds(gbase, RPW)], gidx_v)
        pltpu.sync_copy(emb_hbm.at[gidx_v], grow_v)
        pltpu.sync_copy(grow_v, nf_hbm.at[pl.ds(gbase, RPW)])

        ebase = c * (2 * EPG) + s * EPW
        pltpu.sync_copy(ei_hbm.at[0, pl.ds(ebase, EPW)], src_v)
        pltpu.sync_copy(ei_hbm.at[1, pl.ds(ebase, EPW)], dst_v)
        pltpu.sync_copy(estr_hbm.at[pl.ds(ebase, EPW)], str_v)

        es_off = gid * (NPAD * NPAD)

        @pl.loop(0, 8)
        def _(j):
            @pl.loop(0, 8)
            def _(kk):
                col = kk * 16
                sl = pl.ds(j * 128 + col, 16)
                ei2_v[j, pl.ds(col, 16)] = (src_v[sl] * NPAD + dst_v[sl]
                                            + es_off)
                ev2_v[j, pl.ds(col, 16)] = str_v[sl]

        for cp in zcps:
            cp.wait()
        plsc.subcore_barrier()

        @pl.loop(0, 8)
        def _(j):
            pltpu.sync_copy(ev2_v.at[j], es_hbm.at[ei2_v.at[j]])

    return k(node_ids, edge_index, edge_strength, emb_table)


def _tc_body(nf_ref, srcb_ref, dstb_ref, ts_ref, es_ref, din_ref, dout_ref,
             gt_ref, wq_ref, wk_ref, wv_ref, wo_ref, w1_ref, b1_ref, w2_ref,
             b2_ref, wf_ref, bfv_ref, wp_ref, bp_ref, out_ref):
    f32 = jnp.float32
    dot = functools.partial(jnp.dot, preferred_element_type=f32)

    ones8 = jnp.ones((8, EPG), jnp.bfloat16)
    lane = lax.broadcasted_iota(jnp.int32, (1, 128), 1)

    def counts_col(xb):
        chunks = []
        for cb in range(NPAD // 128):
            oh = (xb == lane + cb * 128).astype(jnp.bfloat16)
            chunks.append(dot(ones8, oh))
        c8 = jnp.concatenate(chunks, axis=1)
        return jnp.transpose(c8)[:, 0:1]

    def bucket_onehot(ccol):
        cf = jnp.maximum(ccol, 1.0)
        lg = (lax.bitcast_convert_type(cf, jnp.int32) >> 23) - 127
        bk = jnp.clip(lg, 0, DEG - 1)
        return (bk == lax.broadcasted_iota(jnp.int32, (1, DEG), 1)
                ).astype(f32)

    deg_in = dot(bucket_onehot(counts_col(dstb_ref[0])), din_ref[...])
    deg_out = dot(bucket_onehot(counts_col(srcb_ref[0])), dout_ref[...])

    h_nodes = nf_ref[0] + deg_in + deg_out
    h = jnp.concatenate(
        [h_nodes, gt_ref[...], jnp.zeros((TPAD - T, D), f32)], axis=0)

    q = dot(h, wq_ref[...])
    km = dot(h, wk_ref[...])
    v = dot(h, wv_ref[...])

    lane = lax.broadcasted_iota(jnp.int32, (1, 128), 1)
    right = jnp.where(lane == 0, 0.0, NEG).astype(f32)
    top = jnp.concatenate(
        [es_ref[0], jnp.broadcast_to(right, (NPAD, 128))], axis=1)
    lane_full = lax.broadcasted_iota(jnp.int32, (1, TPAD), 1)
    bot_row = jnp.where(lane_full <= NPAD, 0.0, NEG).astype(f32)
    bias = jnp.concatenate(
        [top, jnp.broadcast_to(bot_row, (TPAD - NPAD, TPAD))], axis=0)

    scale = f32(1.0 / np.sqrt(DH))
    qb = (q * scale).astype(jnp.bfloat16)
    kb = km.astype(jnp.bfloat16)
    vb = v.astype(jnp.bfloat16)
    tb = bias - jnp.max(bias, axis=1, keepdims=True)
    heads = []
    for i in range(H):
        logits = lax.dot_general(qb[:, i * DH:(i + 1) * DH],
                                 kb[:, i * DH:(i + 1) * DH],
                                 (((1,), (1,)), ((), ())),
                                 preferred_element_type=f32)
        p = jnp.exp(logits + tb)
        sinv = 1.0 / jnp.sum(p, axis=1, keepdims=True)
        heads.append(dot(p.astype(jnp.bfloat16),
                         vb[:, i * DH:(i + 1) * DH]) * sinv)
    o = jnp.concatenate(heads, axis=1)

    o2 = dot(o, wo_ref[...]) + h
    ffh = jnp.maximum(dot(o2, w1_ref[...]) + b1_ref[...], 0.0)
    ffo = dot(ffh, w2_ref[...]) + b2_ref[...] + o2

    nz = jnp.sum((ts_ref[0, 0, :] != 0).astype(jnp.int32))
    ridx = lax.broadcasted_iota(jnp.int32, (TPAD, 1), 0)
    sel = jnp.where(ridx + 1 < nz, ffo, 0.0)
    h0 = (jnp.sum(sel, axis=0, keepdims=True)
          / jnp.maximum(nz, 1).astype(f32))
    hs = h0 + jnp.tanh(dot(h0, wf_ref[...]) + bfv_ref[...])

    graph_rep = ffo[NPAD:NPAD + 1, :]
    s0 = jnp.sum(graph_rep * wp_ref[0:1, :])
    s1 = jnp.sum(hs * wp_ref[1:2, :])
    pred = jnp.maximum(s0 + s1 + bp_ref[0, 0], 0.0)
    out_ref[0, 0, :] = jnp.full((128,), pred, f32)


def _tc_forward(nf, srcb, dstb, ts3, es, deg_in_table, deg_out_table, gt2,
                Wq, Wk, Wv, Wo, W1, b1_2, W2, b2_2, Wf, bf_2, Wp2, bp2):
    full = lambda shape: pl.BlockSpec(shape, lambda b: (0,) * len(shape))
    perb3 = lambda s1_, s2_: pl.BlockSpec((1, s1_, s2_), lambda b: (b, 0, 0))
    out = pl.pallas_call(
        _tc_body,
        grid=(B,),
        in_specs=[
            perb3(NPAD, D),
            perb3(EPG, 128),
            perb3(EPG, 128),
            perb3(1, NPAD),
            perb3(NPAD, NPAD),
            full((DEG, D)),
            full((DEG, D)),
            full((1, D)),
            full((D, D)), full((D, D)), full((D, D)), full((D, D)),
            full((D, FF)), full((1, FF)), full((FF, D)), full((1, D)),
            full((D, D)), full((1, D)),
            full((2, D)),
            full((1, 128)),
        ],
        out_specs=pl.BlockSpec((1, 1, 128), lambda b: (b, 0, 0)),
        out_shape=jax.ShapeDtypeStruct((B, 1, 128), jnp.float32),
        compiler_params=pltpu.CompilerParams(
            dimension_semantics=("parallel",)),
    )(nf, srcb, dstb, ts3, es, deg_in_table, deg_out_table, gt2,
      Wq, Wk, Wv, Wo, W1, b1_2, W2, b2_2, Wf, bf_2, Wp2, bp2)
    return out[:, 0, :1]


def kernel(node_ids, node_timestamp, edge_index, edge_strength, cu_seqlens,
           emb_table, deg_in_table, deg_out_table, graph_token,
           Wq, Wk, Wv, Wo, W1, b1, W2, b2, Wf, bf, Wp, bp):
    node_ids = node_ids.astype(jnp.int32)
    edge_index = edge_index.astype(jnp.int32)

    nf, es = _sc_sparse(node_ids, edge_index, edge_strength, emb_table)

    srcb = jnp.broadcast_to(edge_index[0].reshape(B, EPG, 1), (B, EPG, 128))
    dstb = jnp.broadcast_to(edge_index[1].reshape(B, EPG, 1), (B, EPG, 128))

    return _tc_forward(
        nf.reshape(B, NPAD, D),
        srcb, dstb,
        node_timestamp.reshape(B, 1, NPAD),
        es.reshape(B, NPAD, NPAD),
        deg_in_table, deg_out_table,
        graph_token.reshape(1, D),
        Wq, Wk, Wv, Wo,
        W1, b1.reshape(1, FF), W2, b2.reshape(1, D),
        Wf, bf.reshape(1, D),
        Wp.reshape(2, D),
        jnp.broadcast_to(bp.reshape(1, 1), (1, 128)),
    )
